# Initial kernel scaffold; baseline (speedup 1.0000x reference)
import jax
import jax.numpy as jnp
from jax.experimental import pallas as pl


def kernel(x, edge_index, edge_weights, W1, b1, W2, b2, W3, b3):
    raise NotImplementedError("write your pallas kernel here")


# speedup vs baseline: 4.3386x; 4.3386x over previous
import functools

import jax
import jax.numpy as jnp
from jax import lax
from jax.experimental import pallas as pl
from jax.experimental.pallas import tpu as pltpu
from jax.experimental.pallas import tpu_sc as plsc

N = 10000
E = 160000
NC = 2
NS = 16
L = 16

_mesh = lambda: plsc.VectorSubcoreMesh(core_axis_name="c", subcore_axis_name="s")

BLK = 200
NBLK = N // BLK
BPT = -(-NBLK // NS)


def _make_deg_kernel():
  Wc = 128
  B = 80
  NBATCH = E // B
  NBT = -(-NBATCH // (NC * NS))

  @functools.partial(
      pl.kernel,
      out_type=[jax.ShapeDtypeStruct((N, Wc), jnp.float32)] * NC,
      mesh=_mesh(),
      scratch_types=[
          pltpu.VMEM((B,), jnp.int32),
          pltpu.VMEM((B,), jnp.float32),
          pltpu.VMEM((B, Wc), jnp.float32),
          pltpu.VMEM((BLK, Wc), jnp.float32),
          pltpu.VMEM_SHARED((N, Wc), jnp.float32),
      ],
  )
  def deg_kernel(dst_hbm, w_hbm, out0, out1, didx, wv, rows, zbuf, acc):
    cid = lax.axis_index("c")
    sid = lax.axis_index("s")
    zero16 = jnp.zeros((L,), jnp.float32)

    def zb(i, _):
      for j in range(Wc // L):
        zbuf[i, pl.ds(j * L, L)] = zero16
      return 0
    lax.fori_loop(0, BLK, zb, 0)

    def zr(i, _):
      for j in range(1, Wc // L):
        rows[i, pl.ds(j * L, L)] = zero16
      return 0
    lax.fori_loop(0, B, zr, 0)
    for p in range(BPT):
      blk = sid + p * NS
      @pl.when(blk < NBLK)
      def _():
        pltpu.sync_copy(zbuf, acc.at[pl.ds(blk * BLK, BLK)])
    plsc.subcore_barrier()

    wid = cid * NS + sid

    def batch(t, _):
      gb = t * (NC * NS) + wid
      @pl.when(gb < NBATCH)
      def _():
        off = gb * B
        pltpu.sync_copy(dst_hbm.at[pl.ds(off, B)], didx)
        pltpu.sync_copy(w_hbm.at[pl.ds(off, B)], wv)

        def grp(g, _):
          wg = wv[pl.ds(g * L, L)]
          for i in range(L):
            rows[g * L + i, pl.ds(0, L)] = jnp.broadcast_to(wg[i:i + 1], (L,))
          return 0
        lax.fori_loop(0, B // L, grp, 0)
        pltpu.sync_copy(rows, acc.at[didx], add=True)
      return 0
    lax.fori_loop(0, NBT, batch, 0)
    plsc.subcore_barrier()

    for p in range(BPT):
      blk = sid + p * NS
      r0 = blk * BLK
      @pl.when((blk < NBLK) & (cid == 0))
      def _():
        pltpu.sync_copy(acc.at[pl.ds(r0, BLK)], out0.at[pl.ds(r0, BLK)])
      @pl.when((blk < NBLK) & (cid == 1))
      def _():
        pltpu.sync_copy(acc.at[pl.ds(r0, BLK)], out1.at[pl.ds(r0, BLK)])

  return deg_kernel


def _make_spmm_kernel(C, Wc):
  assert C % NC == 0 and Wc % L == 0
  Cc = C // NC
  B = 80
  EPT = E // NS
  NB = EPT // B

  @functools.partial(
      pl.kernel,
      out_type=[jax.ShapeDtypeStruct((N, Wc), jnp.float32)] * C,
      mesh=_mesh(),
      scratch_types=[
          pltpu.VMEM((B,), jnp.int32),
          pltpu.VMEM((B,), jnp.int32),
          pltpu.VMEM((B,), jnp.float32),
          pltpu.VMEM((B, Wc), jnp.float32),
          pltpu.VMEM((BLK, Wc), jnp.float32),
          pltpu.VMEM_SHARED((N, Wc), jnp.float32),
          pltpu.SemaphoreType.DMA,
      ],
  )
  def spmm_kernel(src_hbm, dst_hbm, w_hbm, *rest):
    tables = rest[:C]
    outs = rest[C:2 * C]
    sidx, didx, wv, rows, zbuf, acc, sem = rest[2 * C:]
    cid = lax.axis_index("c")
    sid = lax.axis_index("s")
    zero16 = jnp.zeros((L,), jnp.float32)

    def zb(i, _):
      for j in range(Wc // L):
        zbuf[i, pl.ds(j * L, L)] = zero16
      return 0
    lax.fori_loop(0, BLK, zb, 0)

    ebase = sid * EPT

    def process(table, out):
      for p in range(BPT):
        blk = sid + p * NS
        @pl.when(blk < NBLK)
        def _():
          pltpu.sync_copy(zbuf, acc.at[pl.ds(blk * BLK, BLK)])
      plsc.subcore_barrier()

      def batch(b, _):
        off = ebase + b * B
        pltpu.sync_copy(src_hbm.at[pl.ds(off, B)], sidx)
        pltpu.sync_copy(dst_hbm.at[pl.ds(off, B)], didx)
        pltpu.sync_copy(w_hbm.at[pl.ds(off, B)], wv)
        pltpu.async_copy(table.at[sidx], rows, sem).wait()

        def grp(g, _):
          wg = wv[pl.ds(g * L, L)]
          for i in range(L):
            wb = jnp.broadcast_to(wg[i:i + 1], (L,))
            for j in range(Wc // L):
              sl = pl.ds(j * L, L)
              rows[g * L + i, sl] = rows[g * L + i, sl] * wb
          return 0
        lax.fori_loop(0, B // L, grp, 0)
        pltpu.sync_copy(rows, acc.at[didx], add=True)
        return 0
      lax.fori_loop(0, NB, batch, 0)
      plsc.subcore_barrier()

      for p in range(BPT):
        blk = sid + p * NS
        @pl.when(blk < NBLK)
        def _():
          r0 = blk * BLK
          pltpu.sync_copy(acc.at[pl.ds(r0, BLK)], out.at[pl.ds(r0, BLK)])
      plsc.subcore_barrier()

    for kc in range(Cc):
      @pl.when(cid == 0)
      def _():
        process(tables[kc], outs[kc])
      @pl.when(cid == 1)
      def _():
        process(tables[Cc + kc], outs[Cc + kc])

  return spmm_kernel


def _make_spmm_split_kernel():
  Wc = 128
  B = 80
  NBATCH = E // B
  NBT = -(-NBATCH // (NC * NS))

  @functools.partial(
      pl.kernel,
      out_type=[jax.ShapeDtypeStruct((N, Wc), jnp.float32)] * NC,
      mesh=_mesh(),
      scratch_types=[
          pltpu.VMEM((B,), jnp.int32),
          pltpu.VMEM((B,), jnp.int32),
          pltpu.VMEM((B,), jnp.float32),
          pltpu.VMEM((B, Wc), jnp.float32),
          pltpu.VMEM((BLK, Wc), jnp.float32),
          pltpu.VMEM_SHARED((N, Wc), jnp.float32),
          pltpu.SemaphoreType.DMA,
      ],
  )
  def spmm_split(src_hbm, dst_hbm, w_hbm, table, out0, out1,
                 sidx, didx, wv, rows, zbuf, acc, sem):
    cid = lax.axis_index("c")
    sid = lax.axis_index("s")
    zero16 = jnp.zeros((L,), jnp.float32)

    def zb(i, _):
      for j in range(Wc // L):
        zbuf[i, pl.ds(j * L, L)] = zero16
      return 0
    lax.fori_loop(0, BLK, zb, 0)

    for p in range(BPT):
      blk = sid + p * NS
      @pl.when(blk < NBLK)
      def _():
        pltpu.sync_copy(zbuf, acc.at[pl.ds(blk * BLK, BLK)])
    plsc.subcore_barrier()

    wid = cid * NS + sid

    def batch(t, _):
      gb = t * (NC * NS) + wid
      @pl.when(gb < NBATCH)
      def _():
        off = gb * B
        pltpu.sync_copy(src_hbm.at[pl.ds(off, B)], sidx)
        pltpu.sync_copy(dst_hbm.at[pl.ds(off, B)], didx)
        pltpu.sync_copy(w_hbm.at[pl.ds(off, B)], wv)
        pltpu.async_copy(table.at[sidx], rows, sem).wait()

        def grp(g, _):
          wg = wv[pl.ds(g * L, L)]
          for i in range(L):
            wb = jnp.broadcast_to(wg[i:i + 1], (L,))
            for j in range(Wc // L):
              sl = pl.ds(j * L, L)
              rows[g * L + i, sl] = rows[g * L + i, sl] * wb
          return 0
        lax.fori_loop(0, B // L, grp, 0)
        pltpu.sync_copy(rows, acc.at[didx], add=True)
      return 0
    lax.fori_loop(0, NBT, batch, 0)
    plsc.subcore_barrier()

    for p in range(BPT):
      blk = sid + p * NS
      r0 = blk * BLK
      @pl.when((blk < NBLK) & (cid == 0))
      def _():
        pltpu.sync_copy(acc.at[pl.ds(r0, BLK)], out0.at[pl.ds(r0, BLK)])
      @pl.when((blk < NBLK) & (cid == 1))
      def _():
        pltpu.sync_copy(acc.at[pl.ds(r0, BLK)], out1.at[pl.ds(r0, BLK)])

  return spmm_split


MB = 1000


def _dis_from_deg(d0, d1):
  deg = d0[:, 0:1] + d1[:, 0:1]
  return jnp.where(deg > 0, lax.rsqrt(deg + 1e-12), 0.0)


def _tc_pre_body(d0, d1, x, xs0, xs1):
  dis = _dis_from_deg(d0[...], d1[...])
  xsc = x[...] * dis
  xs0[...] = xsc[:, :128]
  xs1[...] = xsc[:, 128:]


def _tc_l1_body(d0, d1, p0, p1, w1, b1, *outs):
  dis = _dis_from_deg(d0[...], d1[...])
  s1 = jnp.concatenate([p0[...], p1[...]], axis=1) * dis
  h1 = jnp.maximum(jnp.dot(s1, w1[...], preferred_element_type=jnp.float32)
                   + b1[...], 0.0) * dis
  for k, o in enumerate(outs):
    o[...] = h1[:, k * 128:(k + 1) * 128]


def _tc_l2_body(d0, d1, p0, p1, p2, p3, w2, b2, w3, z0):
  dis = _dis_from_deg(d0[...], d1[...])
  s2 = jnp.concatenate([p0[...], p1[...], p2[...], p3[...]], axis=1) * dis
  h2 = jnp.maximum(jnp.dot(s2, w2[...], preferred_element_type=jnp.float32)
                   + b2[...], 0.0)
  z = jnp.dot(h2, w3[...], preferred_element_type=jnp.float32) * dis
  z0[...] = jnp.concatenate([z, jnp.zeros_like(z)], axis=1)


def _tc_fin_body(d0, d1, p0, p1, b3, x, xnew, gamma_out):
  dis = _dis_from_deg(d0[...], d1[...])
  s3 = (p0[...] + p1[...])[:, :64] * dis
  out3 = s3 + b3[...]
  gamma = 1.0 / (1.0 + jnp.exp(-out3))

  xb = x[...]
  pt = xb[:, :64]
  hch = xb[:, 64:128]
  pmax = xb[:, 128:129]
  mu, pc, lr, eps = 4.0, 1.0, 0.01, 1e-08
  p = pt
  for _ in range(7):
    g = mu * hch / (1.0 + hch * p + eps) - pc
    p = jnp.clip(p + lr * g, 0.0, pmax)
  xnew[...] = pt + gamma * (p - pt)
  gamma_out[...] = gamma


def _row_spec(w):
  return pl.BlockSpec((MB, w), lambda i: (i, 0))


def _full_spec(shape):
  return pl.BlockSpec(shape, lambda i: tuple(0 for _ in shape))


def _tc_call(body, in_specs, out_specs, out_shapes, args):
  return pl.pallas_call(
      body,
      grid=(N // MB,),
      in_specs=in_specs,
      out_specs=out_specs,
      out_shape=out_shapes,
  )(*args)


def kernel(x, edge_index, edge_weights, W1, b1, W2, b2, W3, b3):
  src = edge_index[0].astype(jnp.int32)
  dst = edge_index[1].astype(jnp.int32)
  w = edge_weights.astype(jnp.float32)
  b1r = b1.reshape(1, -1)
  b2r = b2.reshape(1, -1)
  b3r = b3.reshape(1, -1)

  d0, d1 = _make_deg_kernel()(dst, w)

  f32 = jnp.float32
  xs = _tc_call(
      _tc_pre_body,
      [_row_spec(128), _row_spec(128), _row_spec(256)],
      [_row_spec(128), _row_spec(128)],
      [jax.ShapeDtypeStruct((N, 128), f32)] * 2,
      [d0, d1, x],
  )

  p1 = _make_spmm_kernel(2, 128)(src, dst, w, *xs)

  h1s = _tc_call(
      _tc_l1_body,
      [_row_spec(128), _row_spec(128), _row_spec(128), _row_spec(128),
       _full_spec((256, 512)), _full_spec((1, 512))],
      [_row_spec(128)] * 4,
      [jax.ShapeDtypeStruct((N, 128), f32)] * 4,
      [d0, d1, p1[0], p1[1], W1, b1r],
  )

  p2 = _make_spmm_kernel(4, 128)(src, dst, w, *h1s)

  zs = _tc_call(
      _tc_l2_body,
      [_row_spec(128), _row_spec(128)] + [_row_spec(128)] * 4
      + [_full_spec((512, 512)), _full_spec((1, 512)), _full_spec((512, 64))],
      [_row_spec(128)],
      [jax.ShapeDtypeStruct((N, 128), f32)],
      [d0, d1, p2[0], p2[1], p2[2], p2[3], W2, b2r, W3],
  )

  p3 = _make_spmm_split_kernel()(src, dst, w, zs[0])

  x_new, gamma = _tc_call(
      _tc_fin_body,
      [_row_spec(128), _row_spec(128), _row_spec(128), _row_spec(128),
       _full_spec((1, 64)), _row_spec(256)],
      [_row_spec(64), _row_spec(64)],
      [jax.ShapeDtypeStruct((N, 64), f32)] * 2,
      [d0, d1, p3[0], p3[1], b3r, x],
  )
  return (x_new, gamma)


# speedup vs baseline: 7.0450x; 1.6238x over previous
import functools

import jax
import jax.numpy as jnp
from jax import lax
from jax.experimental import pallas as pl
from jax.experimental.pallas import tpu as pltpu
from jax.experimental.pallas import tpu_sc as plsc

N = 10000
E = 160000
NC = 2
NS = 16
L = 16

_mesh = lambda: plsc.VectorSubcoreMesh(core_axis_name="c", subcore_axis_name="s")

BLK = 200
NBLK = N // BLK
BPT = -(-NBLK // NS)


def _make_deg_kernel():
  Wc = 128
  B = 80
  NBATCH = E // B
  NBT = -(-NBATCH // (NC * NS))

  @functools.partial(
      pl.kernel,
      out_type=[jax.ShapeDtypeStruct((N, Wc), jnp.float32)] * NC,
      mesh=_mesh(),
      scratch_types=[
          pltpu.VMEM((B,), jnp.int32),
          pltpu.VMEM((B,), jnp.float32),
          pltpu.VMEM((B, Wc), jnp.float32),
          pltpu.VMEM((BLK, Wc), jnp.float32),
          pltpu.VMEM_SHARED((N, Wc), jnp.float32),
      ],
  )
  def deg_kernel(dst_hbm, w_hbm, out0, out1, didx, wv, rows, zbuf, acc):
    cid = lax.axis_index("c")
    sid = lax.axis_index("s")
    zero16 = jnp.zeros((L,), jnp.float32)

    def zb(i, _):
      for j in range(Wc // L):
        zbuf[i, pl.ds(j * L, L)] = zero16
      return 0
    lax.fori_loop(0, BLK, zb, 0)

    def zr(i, _):
      for j in range(1, Wc // L):
        rows[i, pl.ds(j * L, L)] = zero16
      return 0
    lax.fori_loop(0, B, zr, 0)
    for p in range(BPT):
      blk = sid + p * NS
      @pl.when(blk < NBLK)
      def _():
        pltpu.sync_copy(zbuf, acc.at[pl.ds(blk * BLK, BLK)])
    plsc.subcore_barrier()

    wid = cid * NS + sid

    def batch(t, _):
      gb = t * (NC * NS) + wid
      @pl.when(gb < NBATCH)
      def _():
        off = gb * B
        pltpu.sync_copy(dst_hbm.at[pl.ds(off, B)], didx)
        pltpu.sync_copy(w_hbm.at[pl.ds(off, B)], wv)

        def grp(g, _):
          wg = wv[pl.ds(g * L, L)]
          for i in range(L):
            rows[g * L + i, pl.ds(0, L)] = jnp.broadcast_to(wg[i:i + 1], (L,))
          return 0
        lax.fori_loop(0, B // L, grp, 0)
        pltpu.sync_copy(rows, acc.at[didx], add=True)
      return 0
    lax.fori_loop(0, NBT, batch, 0)
    plsc.subcore_barrier()

    for p in range(BPT):
      blk = sid + p * NS
      r0 = blk * BLK
      @pl.when((blk < NBLK) & (cid == 0))
      def _():
        pltpu.sync_copy(acc.at[pl.ds(r0, BLK)], out0.at[pl.ds(r0, BLK)])
      @pl.when((blk < NBLK) & (cid == 1))
      def _():
        pltpu.sync_copy(acc.at[pl.ds(r0, BLK)], out1.at[pl.ds(r0, BLK)])

  return deg_kernel


def _make_spmm_kernel(C):
  Wc = 128
  B = 128
  NBATCH = E // B
  NT = NS if C >= 2 else NC * NS
  NBT = -(-NBATCH // NT)
  H = (NBT + 1) // 2
  Cc = max(C // NC, 1)
  n_out = C if C >= 2 else NC

  @functools.partial(
      pl.kernel,
      out_type=[jax.ShapeDtypeStruct((N, Wc), jnp.float32)] * n_out,
      mesh=_mesh(),
      scratch_types=[
          pltpu.VMEM((B,), jnp.int32), pltpu.VMEM((B,), jnp.int32),
          pltpu.VMEM((B,), jnp.int32), pltpu.VMEM((B,), jnp.int32),
          pltpu.VMEM((B,), jnp.float32), pltpu.VMEM((B,), jnp.float32),
          pltpu.VMEM((B, Wc), jnp.float32), pltpu.VMEM((B, Wc), jnp.float32),
          pltpu.VMEM((40, Wc), jnp.float32),
          pltpu.VMEM_SHARED((N, Wc), jnp.float32),
          pltpu.SemaphoreType.DMA, pltpu.SemaphoreType.DMA,
      ],
  )
  def spmm_kernel(src_hbm, dst_hbm, w_hbm, *rest):
    tables = rest[:C]
    outs = rest[C:C + n_out]
    (sidx0, sidx1, didx0, didx1, wv0, wv1, rows0, rows1,
     zbuf, acc, sem0, sem1) = rest[C + n_out:]
    sidx = (sidx0, sidx1)
    didx = (didx0, didx1)
    wv = (wv0, wv1)
    rows = (rows0, rows1)
    sem = (sem0, sem1)
    cid = lax.axis_index("c")
    sid = lax.axis_index("s")
    tid = sid if C >= 2 else cid * NS + sid
    zero16 = jnp.zeros((L,), jnp.float32)

    def zb(i, _):
      for j in range(Wc // L):
        zbuf[i, pl.ds(j * L, L)] = zero16
      return 0
    lax.fori_loop(0, 40, zb, 0)

    def process(table, out):
      for p in range(BPT):
        blk = sid + p * NS
        @pl.when(blk < NBLK)
        def _():
          for q in range(BLK // 40):
            pltpu.sync_copy(zbuf, acc.at[pl.ds(blk * BLK + q * 40, 40)])
      plsc.subcore_barrier()

      def fire(k, t):
        gb = t * NT + tid
        @pl.when(gb < NBATCH)
        def _():
          off = gb * B
          pltpu.sync_copy(src_hbm.at[pl.ds(off, B)], sidx[k])
          pltpu.sync_copy(dst_hbm.at[pl.ds(off, B)], didx[k])
          pltpu.sync_copy(w_hbm.at[pl.ds(off, B)], wv[k])
          pltpu.async_copy(table.at[sidx[k]], rows[k], sem[k])

      def consume(k, t):
        gb = t * NT + tid
        @pl.when(gb < NBATCH)
        def _():
          pltpu.make_async_copy(table.at[sidx[k]], rows[k], sem[k]).wait()
          rk = rows[k]
          wk = wv[k]

          def grp(g, _):
            wg = wk[pl.ds(g * L, L)]
            for i in range(L):
              wb = jnp.broadcast_to(wg[i:i + 1], (L,))
              for j in range(Wc // L):
                sl = pl.ds(j * L, L)
                rk[g * L + i, sl] = rk[g * L + i, sl] * wb
            return 0
          lax.fori_loop(0, B // L, grp, 0)
          pltpu.sync_copy(rk, acc.at[didx[k]], add=True)

      fire(0, 0)

      def dbl(u, _):
        t0 = 2 * u
        fire(1, t0 + 1)
        consume(0, t0)
        fire(0, t0 + 2)
        consume(1, t0 + 1)
        return 0
      lax.fori_loop(0, H, dbl, 0)
      plsc.subcore_barrier()

      for p in range(BPT):
        blk = sid + p * NS
        @pl.when(blk < NBLK)
        def _():
          r0 = blk * BLK
          pltpu.sync_copy(acc.at[pl.ds(r0, BLK)], out.at[pl.ds(r0, BLK)])
      plsc.subcore_barrier()

    if C >= 2:
      for kc in range(Cc):
        @pl.when(cid == 0)
        def _():
          process(tables[kc], outs[kc])
        @pl.when(cid == 1)
        def _():
          process(tables[Cc + kc], outs[Cc + kc])
    else:
      @pl.when(cid == 0)
      def _():
        process(tables[0], outs[0])
      @pl.when(cid == 1)
      def _():
        process(tables[0], outs[1])

  return spmm_kernel


MB = 1000


def _dis_from_deg(d0, d1):
  deg = d0[:, 0:1] + d1[:, 0:1]
  return jnp.where(deg > 0, lax.rsqrt(deg + 1e-12), 0.0)


def _tc_pre_body(d0, d1, x, xs0, xs1):
  dis = _dis_from_deg(d0[...], d1[...])
  xsc = x[...] * dis
  xs0[...] = xsc[:, :128]
  xs1[...] = xsc[:, 128:]


def _tc_l1_body(d0, d1, p0, p1, w1, b1, *outs):
  dis = _dis_from_deg(d0[...], d1[...])
  s1 = jnp.concatenate([p0[...], p1[...]], axis=1) * dis
  h1 = jnp.maximum(jnp.dot(s1, w1[...], preferred_element_type=jnp.float32)
                   + b1[...], 0.0) * dis
  for k, o in enumerate(outs):
    o[...] = h1[:, k * 128:(k + 1) * 128]


def _tc_l2_body(d0, d1, p0, p1, p2, p3, w2, b2, w3, z0):
  dis = _dis_from_deg(d0[...], d1[...])
  s2 = jnp.concatenate([p0[...], p1[...], p2[...], p3[...]], axis=1) * dis
  h2 = jnp.maximum(jnp.dot(s2, w2[...], preferred_element_type=jnp.float32)
                   + b2[...], 0.0)
  z = jnp.dot(h2, w3[...], preferred_element_type=jnp.float32) * dis
  z0[...] = jnp.concatenate([z, jnp.zeros_like(z)], axis=1)


def _tc_fin_body(d0, d1, p0, p1, b3, x, xnew, gamma_out):
  dis = _dis_from_deg(d0[...], d1[...])
  s3 = (p0[...] + p1[...])[:, :64] * dis
  out3 = s3 + b3[...]
  gamma = 1.0 / (1.0 + jnp.exp(-out3))

  xb = x[...]
  pt = xb[:, :64]
  hch = xb[:, 64:128]
  pmax = xb[:, 128:129]
  mu, pc, lr, eps = 4.0, 1.0, 0.01, 1e-08
  p = pt
  for _ in range(7):
    g = mu * hch / (1.0 + hch * p + eps) - pc
    p = jnp.clip(p + lr * g, 0.0, pmax)
  xnew[...] = pt + gamma * (p - pt)
  gamma_out[...] = gamma


def _row_spec(w):
  return pl.BlockSpec((MB, w), lambda i: (i, 0))


def _full_spec(shape):
  return pl.BlockSpec(shape, lambda i: tuple(0 for _ in shape))


def _tc_call(body, in_specs, out_specs, out_shapes, args):
  return pl.pallas_call(
      body,
      grid=(N // MB,),
      in_specs=in_specs,
      out_specs=out_specs,
      out_shape=out_shapes,
  )(*args)


def kernel(x, edge_index, edge_weights, W1, b1, W2, b2, W3, b3):
  src = edge_index[0].astype(jnp.int32)
  dst = edge_index[1].astype(jnp.int32)
  w = edge_weights.astype(jnp.float32)
  b1r = b1.reshape(1, -1)
  b2r = b2.reshape(1, -1)
  b3r = b3.reshape(1, -1)

  d0, d1 = _make_deg_kernel()(dst, w)

  f32 = jnp.float32
  xs = _tc_call(
      _tc_pre_body,
      [_row_spec(128), _row_spec(128), _row_spec(256)],
      [_row_spec(128), _row_spec(128)],
      [jax.ShapeDtypeStruct((N, 128), f32)] * 2,
      [d0, d1, x],
  )

  p1 = _make_spmm_kernel(2)(src, dst, w, *xs)

  h1s = _tc_call(
      _tc_l1_body,
      [_row_spec(128), _row_spec(128), _row_spec(128), _row_spec(128),
       _full_spec((256, 512)), _full_spec((1, 512))],
      [_row_spec(128)] * 4,
      [jax.ShapeDtypeStruct((N, 128), f32)] * 4,
      [d0, d1, p1[0], p1[1], W1, b1r],
  )

  p2 = _make_spmm_kernel(4)(src, dst, w, *h1s)

  zs = _tc_call(
      _tc_l2_body,
      [_row_spec(128), _row_spec(128)] + [_row_spec(128)] * 4
      + [_full_spec((512, 512)), _full_spec((1, 512)), _full_spec((512, 64))],
      [_row_spec(128)],
      [jax.ShapeDtypeStruct((N, 128), f32)],
      [d0, d1, p2[0], p2[1], p2[2], p2[3], W2, b2r, W3],
  )

  p3 = _make_spmm_kernel(1)(src, dst, w, zs[0])

  x_new, gamma = _tc_call(
      _tc_fin_body,
      [_row_spec(128), _row_spec(128), _row_spec(128), _row_spec(128),
       _full_spec((1, 64)), _row_spec(256)],
      [_row_spec(64), _row_spec(64)],
      [jax.ShapeDtypeStruct((N, 64), f32)] * 2,
      [d0, d1, p3[0], p3[1], b3r, x],
  )
  return (x_new, gamma)


# speedup vs baseline: 8.1535x; 1.1573x over previous
import functools

import jax
import jax.numpy as jnp
from jax import lax
from jax.experimental import pallas as pl
from jax.experimental.pallas import tpu as pltpu
from jax.experimental.pallas import tpu_sc as plsc

N = 10000
E = 160000
NC = 2
NS = 16
L = 16

_mesh = lambda: plsc.VectorSubcoreMesh(core_axis_name="c", subcore_axis_name="s")

BLK = 200
NBLK = N // BLK
BPT = -(-NBLK // NS)


def _make_deg_kernel():
  Wc = 128
  B = 128
  NBATCH = E // B
  NT = NC * NS
  NBT = -(-NBATCH // NT)
  H = (NBT + 1) // 2

  @functools.partial(
      pl.kernel,
      out_type=[jax.ShapeDtypeStruct((N, Wc), jnp.float32)] * NC,
      mesh=_mesh(),
      scratch_types=[
          pltpu.VMEM((8, B), jnp.int32), pltpu.VMEM((8, B), jnp.int32),
          pltpu.VMEM((B,), jnp.float32), pltpu.VMEM((B,), jnp.float32),
          pltpu.VMEM((B, Wc), jnp.float32), pltpu.VMEM((B, Wc), jnp.float32),
          pltpu.VMEM((40, Wc), jnp.float32),
          pltpu.VMEM_SHARED((N, Wc), jnp.float32),
          pltpu.SemaphoreType.DMA, pltpu.SemaphoreType.DMA,
      ],
  )
  def deg_kernel(pk_hbm, w_hbm, out0, out1, ib0, ib1, wv0, wv1,
                 rows0, rows1, zbuf, acc, ss0, ss1):
    ibuf = (ib0, ib1)
    wv = (wv0, wv1)
    rows = (rows0, rows1)
    ssem = (ss0, ss1)
    cid = lax.axis_index("c")
    sid = lax.axis_index("s")
    tid = cid * NS + sid
    zero16 = jnp.zeros((L,), jnp.float32)

    def zb(i, _):
      for j in range(Wc // L):
        zbuf[i, pl.ds(j * L, L)] = zero16
      return 0
    lax.fori_loop(0, 40, zb, 0)

    def zr(i, _):
      for j in range(1, Wc // L):
        rows0[i, pl.ds(j * L, L)] = zero16
        rows1[i, pl.ds(j * L, L)] = zero16
      return 0
    lax.fori_loop(0, B, zr, 0)

    for p in range(BPT):
      blk = sid + p * NS
      @pl.when(blk < NBLK)
      def _():
        for q in range(BLK // 40):
          pltpu.sync_copy(zbuf, acc.at[pl.ds(blk * BLK + q * 40, 40)])
    plsc.subcore_barrier()

    def step(k, t):
      gb = t * NT + tid
      @pl.when((t >= 2) & ((t - 2) * NT + tid < NBATCH))
      def _():
        pltpu.make_async_copy(rows[k], acc.at[ibuf[k].at[1]], ssem[k]).wait()
      @pl.when(gb < NBATCH)
      def _():
        pltpu.sync_copy(pk_hbm.at[gb], ibuf[k])
        pltpu.sync_copy(w_hbm.at[pl.ds(gb * B, B)], wv[k])
        rk = rows[k]
        ik = ibuf[k]
        wk = wv[k]

        def grp(g, _):
          wg = wk[pl.ds(g * L, L)]
          for i in range(L):
            rk[g * L + i, pl.ds(0, L)] = jnp.broadcast_to(wg[i:i + 1], (L,))
          return 0
        lax.fori_loop(0, B // L, grp, 0)
        pltpu.async_copy(rk, acc.at[ik.at[1]], ssem[k], add=True)

    def dbl(u, _):
      t0 = 2 * u
      step(0, t0)
      step(1, t0 + 1)
      return 0
    lax.fori_loop(0, H, dbl, 0)

    for k, tl in ((0, 2 * H - 2), (1, 2 * H - 1)):
      @pl.when(tl * NT + tid < NBATCH)
      def _():
        pltpu.make_async_copy(rows[k], acc.at[ibuf[k].at[1]], ssem[k]).wait()
    plsc.subcore_barrier()

    for p in range(BPT):
      blk = sid + p * NS
      r0 = blk * BLK
      @pl.when((blk < NBLK) & (cid == 0))
      def _():
        pltpu.sync_copy(acc.at[pl.ds(r0, BLK)], out0.at[pl.ds(r0, BLK)])
      @pl.when((blk < NBLK) & (cid == 1))
      def _():
        pltpu.sync_copy(acc.at[pl.ds(r0, BLK)], out1.at[pl.ds(r0, BLK)])

  return deg_kernel


def _make_spmm_kernel(C):
  Wc = 128
  B = 128
  NBATCH = E // B
  NT = NS if C >= 2 else NC * NS
  NBT = -(-NBATCH // NT)
  H = (NBT + 1) // 2
  Cc = max(C // NC, 1)
  n_out = C if C >= 2 else NC

  @functools.partial(
      pl.kernel,
      out_type=[jax.ShapeDtypeStruct((N, Wc), jnp.float32)] * n_out,
      mesh=_mesh(),
      scratch_types=[
          pltpu.VMEM((8, B), jnp.int32), pltpu.VMEM((8, B), jnp.int32),
          pltpu.VMEM((B,), jnp.float32), pltpu.VMEM((B,), jnp.float32),
          pltpu.VMEM((B, Wc), jnp.float32), pltpu.VMEM((B, Wc), jnp.float32),
          pltpu.VMEM((40, Wc), jnp.float32),
          pltpu.VMEM_SHARED((N, Wc), jnp.float32),
          pltpu.SemaphoreType.DMA, pltpu.SemaphoreType.DMA,
          pltpu.SemaphoreType.DMA, pltpu.SemaphoreType.DMA,
      ],
  )
  def spmm_kernel(pk_hbm, w_hbm, *rest):
    tables = rest[:C]
    outs = rest[C:C + n_out]
    (ib0, ib1, wv0, wv1, rows0, rows1, zbuf, acc,
     gs0, gs1, ss0, ss1) = rest[C + n_out:]
    ibuf = (ib0, ib1)
    wv = (wv0, wv1)
    rows = (rows0, rows1)
    gsem = (gs0, gs1)
    ssem = (ss0, ss1)
    cid = lax.axis_index("c")
    sid = lax.axis_index("s")
    tid = sid if C >= 2 else cid * NS + sid
    zero16 = jnp.zeros((L,), jnp.float32)

    def zb(i, _):
      for j in range(Wc // L):
        zbuf[i, pl.ds(j * L, L)] = zero16
      return 0
    lax.fori_loop(0, 40, zb, 0)

    def process(table, out):
      for p in range(BPT):
        blk = sid + p * NS
        @pl.when(blk < NBLK)
        def _():
          for q in range(BLK // 40):
            pltpu.sync_copy(zbuf, acc.at[pl.ds(blk * BLK + q * 40, 40)])
      plsc.subcore_barrier()

      def fire(k, t):
        gb = t * NT + tid
        @pl.when(gb < NBATCH)
        def _():
          pltpu.sync_copy(pk_hbm.at[gb], ibuf[k])
          pltpu.sync_copy(w_hbm.at[pl.ds(gb * B, B)], wv[k])
          pltpu.async_copy(table.at[ibuf[k].at[0]], rows[k], gsem[k])

      def consume(k, t):
        gb = t * NT + tid
        @pl.when(gb < NBATCH)
        def _():
          pltpu.make_async_copy(table.at[ibuf[k].at[0]], rows[k],
                                gsem[k]).wait()
          rk = rows[k]
          ik = ibuf[k]
          wk = wv[k]

          def grp(g, _):
            wg = wk[pl.ds(g * L, L)]
            for i in range(L):
              wb = jnp.broadcast_to(wg[i:i + 1], (L,))
              for j in range(Wc // L):
                sl = pl.ds(j * L, L)
                rk[g * L + i, sl] = rk[g * L + i, sl] * wb
            return 0
          lax.fori_loop(0, B // L, grp, 0)
          pltpu.sync_copy(rk, acc.at[ik.at[1]], add=True)

      fire(0, 0)

      def dbl(u, _):
        t0 = 2 * u
        fire(1, t0 + 1)
        consume(0, t0)
        fire(0, t0 + 2)
        consume(1, t0 + 1)
        return 0
      lax.fori_loop(0, H, dbl, 0)
      plsc.subcore_barrier()

      for p in range(BPT):
        blk = sid + p * NS
        @pl.when(blk < NBLK)
        def _():
          r0 = blk * BLK
          pltpu.sync_copy(acc.at[pl.ds(r0, BLK)], out.at[pl.ds(r0, BLK)])
      plsc.subcore_barrier()

    if C >= 2:
      for kc in range(Cc):
        @pl.when(cid == 0)
        def _():
          process(tables[kc], outs[kc])
        @pl.when(cid == 1)
        def _():
          process(tables[Cc + kc], outs[Cc + kc])
    else:
      @pl.when(cid == 0)
      def _():
        process(tables[0], outs[0])
      @pl.when(cid == 1)
      def _():
        process(tables[0], outs[1])

  return spmm_kernel


MB = 1000


def _dis_from_deg(d0, d1):
  deg = d0[:, 0:1] + d1[:, 0:1]
  return jnp.where(deg > 0, lax.rsqrt(deg + 1e-12), 0.0)


def _tc_pre_body(d0, d1, x, xs0, xs1):
  dis = _dis_from_deg(d0[...], d1[...])
  xsc = x[...] * dis
  xs0[...] = xsc[:, :128]
  xs1[...] = xsc[:, 128:]


def _tc_l1_body(d0, d1, p0, p1, w1, b1, *outs):
  dis = _dis_from_deg(d0[...], d1[...])
  s1 = jnp.concatenate([p0[...], p1[...]], axis=1) * dis
  h1 = jnp.maximum(jnp.dot(s1, w1[...], preferred_element_type=jnp.float32)
                   + b1[...], 0.0) * dis
  for k, o in enumerate(outs):
    o[...] = h1[:, k * 128:(k + 1) * 128]


def _tc_l2_body(d0, d1, p0, p1, p2, p3, w2, b2, w3, z0):
  dis = _dis_from_deg(d0[...], d1[...])
  s2 = jnp.concatenate([p0[...], p1[...], p2[...], p3[...]], axis=1) * dis
  h2 = jnp.maximum(jnp.dot(s2, w2[...], preferred_element_type=jnp.float32)
                   + b2[...], 0.0)
  z = jnp.dot(h2, w3[...], preferred_element_type=jnp.float32) * dis
  z0[...] = jnp.concatenate([z, jnp.zeros_like(z)], axis=1)


def _tc_fin_body(d0, d1, p0, p1, b3, x, xnew, gamma_out):
  dis = _dis_from_deg(d0[...], d1[...])
  s3 = (p0[...] + p1[...])[:, :64] * dis
  out3 = s3 + b3[...]
  gamma = 1.0 / (1.0 + jnp.exp(-out3))

  xb = x[...]
  pt = xb[:, :64]
  hch = xb[:, 64:128]
  pmax = xb[:, 128:129]
  mu, pc, lr, eps = 4.0, 1.0, 0.01, 1e-08
  p = pt
  for _ in range(7):
    g = mu * hch / (1.0 + hch * p + eps) - pc
    p = jnp.clip(p + lr * g, 0.0, pmax)
  xnew[...] = pt + gamma * (p - pt)
  gamma_out[...] = gamma


def _row_spec(w):
  return pl.BlockSpec((MB, w), lambda i: (i, 0))


def _full_spec(shape):
  return pl.BlockSpec(shape, lambda i: tuple(0 for _ in shape))


def _tc_call(body, in_specs, out_specs, out_shapes, args):
  return pl.pallas_call(
      body,
      grid=(N // MB,),
      in_specs=in_specs,
      out_specs=out_specs,
      out_shape=out_shapes,
  )(*args)


def _pack_edges(src, dst):
  B = 128
  nb = E // B
  p = jnp.stack([src.reshape(nb, B), dst.reshape(nb, B)], axis=1)
  return jnp.concatenate(
      [p, jnp.zeros((nb, 6, B), jnp.int32)], axis=1)


def kernel(x, edge_index, edge_weights, W1, b1, W2, b2, W3, b3):
  src = edge_index[0].astype(jnp.int32)
  dst = edge_index[1].astype(jnp.int32)
  w = edge_weights.astype(jnp.float32)
  b1r = b1.reshape(1, -1)
  b2r = b2.reshape(1, -1)
  b3r = b3.reshape(1, -1)
  pk = _pack_edges(src, dst)

  d0, d1 = _make_deg_kernel()(pk, w)

  f32 = jnp.float32
  xs = _tc_call(
      _tc_pre_body,
      [_row_spec(128), _row_spec(128), _row_spec(256)],
      [_row_spec(128), _row_spec(128)],
      [jax.ShapeDtypeStruct((N, 128), f32)] * 2,
      [d0, d1, x],
  )

  p1 = _make_spmm_kernel(2)(pk, w, *xs)

  h1s = _tc_call(
      _tc_l1_body,
      [_row_spec(128), _row_spec(128), _row_spec(128), _row_spec(128),
       _full_spec((256, 512)), _full_spec((1, 512))],
      [_row_spec(128)] * 4,
      [jax.ShapeDtypeStruct((N, 128), f32)] * 4,
      [d0, d1, p1[0], p1[1], W1, b1r],
  )

  p2 = _make_spmm_kernel(4)(pk, w, *h1s)

  zs = _tc_call(
      _tc_l2_body,
      [_row_spec(128), _row_spec(128)] + [_row_spec(128)] * 4
      + [_full_spec((512, 512)), _full_spec((1, 512)), _full_spec((512, 64))],
      [_row_spec(128)],
      [jax.ShapeDtypeStruct((N, 128), f32)],
      [d0, d1, p2[0], p2[1], p2[2], p2[3], W2, b2r, W3],
  )

  p3 = _make_spmm_kernel(1)(pk, w, zs[0])

  x_new, gamma = _tc_call(
      _tc_fin_body,
      [_row_spec(128), _row_spec(128), _row_spec(128), _row_spec(128),
       _full_spec((1, 64)), _row_spec(256)],
      [_row_spec(64), _row_spec(64)],
      [jax.ShapeDtypeStruct((N, 64), f32)] * 2,
      [d0, d1, p3[0], p3[1], b3r, x],
  )
  return (x_new, gamma)
